# Initial kernel scaffold; baseline (speedup 1.0000x reference)
import jax
import jax.numpy as jnp
from jax.experimental import pallas as pl


def kernel(x, edge_index, edge_attr, batch, params):
    raise NotImplementedError("write your pallas kernel here")


# speedup vs baseline: 6.6183x; 6.6183x over previous
import functools
import math

import jax
import jax.numpy as jnp
import numpy as np
from jax import lax
from jax.experimental import pallas as pl
from jax.experimental.pallas import tpu as pltpu
from jax.experimental.pallas import tpu_sc as plsc

F32 = jnp.float32
I32 = jnp.int32

N = 10000
G = 128
HEADS = 8
HID = 128
S1 = 75
S1P = 80
D_XL = HEADS * HID
RB = 256
NP = 41 * RB
NC, NS, NWK = 2, 16, 32
CHUNK = 128
AGG_B = 64
EL = 160000 + N
EP = NWK * CHUNK * math.ceil(EL / (NWK * CHUNK))
GT_ROWS = 256
BP = NWK * CHUNK * math.ceil(N / (NWK * CHUNK))
ISQ = 1.0 / math.sqrt(float(HID))
CH = 256


def _sc_mesh():
    return plsc.VectorSubcoreMesh(core_axis_name="c", subcore_axis_name="s")


def _make_count(n_rows, e_pad):
    ew = e_pad // NWK
    nch = ew // CHUNK
    stripe = n_rows // NS

    @functools.partial(
        pl.kernel,
        out_type=jax.ShapeDtypeStruct((NC, n_rows, 16), F32),
        mesh=_sc_mesh(),
        scratch_types=[
            pltpu.VMEM((CHUNK,), I32),
            pltpu.VMEM((CHUNK, 16), F32),
            pltpu.VMEM_SHARED((n_rows, 16), F32),
            pltpu.SemaphoreType.DMA,
        ],
    )
    def k(didx_hbm, zeros_hbm, ones_hbm, out_hbm, didx, ones_v, acc_sh, sem):
        cid = lax.axis_index("c")
        sid = lax.axis_index("s")
        base = (cid * NS + sid) * ew
        pltpu.sync_copy(zeros_hbm.at[pl.ds(sid * stripe, stripe)],
                        acc_sh.at[pl.ds(sid * stripe, stripe)])
        pltpu.sync_copy(ones_hbm, ones_v)
        plsc.subcore_barrier()

        def chunk(c, carry):
            cb = base + c * CHUNK
            pltpu.sync_copy(didx_hbm.at[pl.ds(cb, CHUNK)], didx)
            pltpu.sync_copy(ones_v, acc_sh.at[didx], add=True)
            return carry

        lax.fori_loop(0, nch, chunk, 0)
        plsc.subcore_barrier()
        pltpu.sync_copy(acc_sh.at[pl.ds(sid * stripe, stripe)],
                        out_hbm.at[cid, pl.ds(sid * stripe, stripe)])

    return k


def _make_gather(n_rows, e_pad):
    ew = e_pad // NWK
    nch = ew // CHUNK

    @functools.partial(
        pl.kernel,
        out_type=jax.ShapeDtypeStruct((e_pad, 128), F32),
        mesh=_sc_mesh(),
        scratch_types=[
            pltpu.VMEM((CHUNK,), I32),
            pltpu.VMEM((CHUNK, 128), F32),
            pltpu.SemaphoreType.DMA,
        ],
    )
    def k(tab_hbm, idx_hbm, out_hbm, idx, rows, sem):
        cid = lax.axis_index("c")
        sid = lax.axis_index("s")
        base = (cid * NS + sid) * ew

        @pl.loop(0, nch)
        def chunk(c):
            cb = base + c * CHUNK
            pltpu.sync_copy(idx_hbm.at[pl.ds(cb, CHUNK)], idx)
            pltpu.async_copy(tab_hbm.at[idx], rows, sem).wait()
            pltpu.sync_copy(rows, out_hbm.at[pl.ds(cb, CHUNK)])

    return k


def _make_scatter(n_rows, e_pad, W):
    ew = e_pad // NWK
    nch = ew // CHUNK
    stripe = n_rows // NS

    @functools.partial(
        pl.kernel,
        out_type=jax.ShapeDtypeStruct((NC, n_rows, W), F32),
        mesh=_sc_mesh(),
        scratch_types=[
            pltpu.VMEM((CHUNK,), I32),
            pltpu.VMEM((CHUNK, W), F32),
            pltpu.VMEM_SHARED((n_rows, W), F32),
            pltpu.SemaphoreType.DMA,
        ],
    )
    def k(val_hbm, idx_hbm, zeros_hbm, out_hbm, idx, vals, acc_sh, sem):
        cid = lax.axis_index("c")
        sid = lax.axis_index("s")
        base = (cid * NS + sid) * ew
        pltpu.sync_copy(zeros_hbm.at[pl.ds(sid * stripe, stripe)],
                        acc_sh.at[pl.ds(sid * stripe, stripe)])
        plsc.subcore_barrier()

        @pl.loop(0, nch)
        def chunk(c):
            cb = base + c * CHUNK
            pltpu.sync_copy(idx_hbm.at[pl.ds(cb, CHUNK)], idx)
            pltpu.sync_copy(val_hbm.at[pl.ds(cb, CHUNK)], vals)
            pltpu.sync_copy(vals, acc_sh.at[idx], add=True)

        plsc.subcore_barrier()
        pltpu.sync_copy(acc_sh.at[pl.ds(sid * stripe, stripe)],
                        out_hbm.at[cid, pl.ds(sid * stripe, stripe)])

    return k


EB = 1024


def _edge_ex(als, ald, Ms, Md):
    def body(a_ref, b_ref, ms_ref, md_ref, o_ref):
        a = a_ref[...] + b_ref[...]
        a = jnp.where(a > 0, a, 0.2 * a)
        mvec = ms_ref[0:1, :] + md_ref[0:1, :]
        o_ref[...] = jnp.exp(a - mvec)[:, :16]

    return pl.pallas_call(
        body, grid=(EP // EB,),
        in_specs=[pl.BlockSpec((EB, 128), lambda i: (i, 0)),
                  pl.BlockSpec((EB, 128), lambda i: (i, 0)),
                  pl.BlockSpec((8, 128), lambda i: (0, 0)),
                  pl.BlockSpec((8, 128), lambda i: (0, 0))],
        out_specs=pl.BlockSpec((EB, 16), lambda i: (i, 0)),
        out_shape=jax.ShapeDtypeStruct((EP, 16), F32))(als, ald, Ms, Md)


def _edge_div(ex16, deng):
    def body(e_ref, d_ref, o_ref):
        o_ref[...] = e_ref[...] / d_ref[...][:, :16]

    return pl.pallas_call(
        body, grid=(EP // EB,),
        in_specs=[pl.BlockSpec((EB, 16), lambda i: (i, 0)),
                  pl.BlockSpec((EB, 128), lambda i: (i, 0))],
        out_specs=pl.BlockSpec((EB, 16), lambda i: (i, 0)),
        out_shape=jax.ShapeDtypeStruct((EP, 16), F32))(ex16, deng)


def _edge_mul(ga, gb):
    def body(a_ref, b_ref, o_ref):
        o_ref[...] = (a_ref[...] * b_ref[...])[:, :16]

    return pl.pallas_call(
        body, grid=(EP // EB,),
        in_specs=[pl.BlockSpec((EB, 128), lambda i: (i, 0)),
                  pl.BlockSpec((EB, 128), lambda i: (i, 0))],
        out_specs=pl.BlockSpec((EB, 16), lambda i: (i, 0)),
        out_shape=jax.ShapeDtypeStruct((EP, 16), F32))(ga, gb)


def _edge_msg(w16, xgs, nheads):
    def body(*refs):
        w_ref = refs[0]
        xg_refs = refs[1:1 + nheads]
        lo_ref, hi_ref = refs[1 + nheads:]
        acc = jnp.zeros((EB, 128), F32)
        for h in range(nheads):
            acc = acc + w_ref[...][:, h:h + 1] * xg_refs[h][...]
        lo_ref[...] = acc[:, :64]
        hi_ref[...] = acc[:, 64:]

    espec = pl.BlockSpec((EB, 128), lambda i: (i, 0))
    return pl.pallas_call(
        body, grid=(EP // EB,),
        in_specs=[pl.BlockSpec((EB, 16), lambda i: (i, 0))] + [espec] * nheads,
        out_specs=[pl.BlockSpec((EB, 64), lambda i: (i, 0)),
                   pl.BlockSpec((EB, 64), lambda i: (i, 0))],
        out_shape=[jax.ShapeDtypeStruct((EP, 64), F32),
                   jax.ShapeDtypeStruct((EP, 64), F32)])(w16, *xgs)


def _mm(x, W, b=None, act=None):
    NPr, K = x.shape
    M = W.shape[1]
    has_b = b is not None

    def body(*refs):
        if has_b:
            x_ref, w_ref, b_ref, o_ref = refs
        else:
            x_ref, w_ref, o_ref = refs
        acc = jnp.dot(x_ref[...], w_ref[...], preferred_element_type=F32)
        if has_b:
            acc = acc + b_ref[...]
        if act == "relu":
            acc = jnp.maximum(acc, 0.0)
        o_ref[...] = acc

    in_specs = [pl.BlockSpec((RB, K), lambda i: (i, 0)),
                pl.BlockSpec((K, M), lambda i: (0, 0))]
    ins = [x, W]
    if has_b:
        in_specs.append(pl.BlockSpec((1, M), lambda i: (0, 0)))
        ins.append(b.reshape(1, M))
    return pl.pallas_call(
        body, grid=(NPr // RB,), in_specs=in_specs,
        out_specs=pl.BlockSpec((RB, M), lambda i: (i, 0)),
        out_shape=jax.ShapeDtypeStruct((NPr, M), F32))(*ins)


def _gat_prep(x, WAs, WAd):
    def body(x_ref, was_ref, wad_ref, as_ref, ad_ref, ms_ref, md_ref):
        i = pl.program_id(0)
        xv = x_ref[...]
        a_s = jnp.dot(xv, was_ref[...], preferred_element_type=F32)
        a_d = jnp.dot(xv, wad_ref[...], preferred_element_type=F32)
        as_ref[...] = a_s
        ad_ref[...] = a_d

        @pl.when(i == 0)
        def _():
            ms_ref[...] = jnp.zeros_like(ms_ref)
            md_ref[...] = jnp.zeros_like(md_ref)

        ms_ref[...] = jnp.maximum(
            ms_ref[...],
            jnp.broadcast_to(a_s.max(axis=0, keepdims=True), ms_ref.shape))
        md_ref[...] = jnp.maximum(
            md_ref[...],
            jnp.broadcast_to(a_d.max(axis=0, keepdims=True), md_ref.shape))

    return pl.pallas_call(
        body, grid=(NP // RB,),
        in_specs=[pl.BlockSpec((RB, HID), lambda i: (i, 0)),
                  pl.BlockSpec((HID, 128), lambda i: (0, 0)),
                  pl.BlockSpec((HID, 128), lambda i: (0, 0))],
        out_specs=[pl.BlockSpec((RB, 128), lambda i: (i, 0)),
                   pl.BlockSpec((RB, 128), lambda i: (i, 0)),
                   pl.BlockSpec((8, 128), lambda i: (0, 0)),
                   pl.BlockSpec((8, 128), lambda i: (0, 0))],
        out_shape=[jax.ShapeDtypeStruct((NP, 128), F32),
                   jax.ShapeDtypeStruct((NP, 128), F32),
                   jax.ShapeDtypeStruct((8, 128), F32),
                   jax.ShapeDtypeStruct((8, 128), F32)])(x, WAs, WAd)


def _combine128(pa, pb, b, scale, relu, mask_rows):
    def body(pa_ref, pb_ref, b_ref, o_ref):
        i = pl.program_id(0)
        v = jnp.concatenate([pa_ref[0] + pa_ref[1], pb_ref[0] + pb_ref[1]],
                            axis=1) * scale + b_ref[...]
        if relu:
            v = jnp.maximum(v, 0.0)
        if mask_rows:
            rid = i * RB + lax.broadcasted_iota(I32, (RB, 1), 0)
            v = jnp.where(rid < N, v, 0.0)
        o_ref[...] = v

    return pl.pallas_call(
        body, grid=(NP // RB,),
        in_specs=[pl.BlockSpec((NC, RB, 64), lambda i: (0, i, 0)),
                  pl.BlockSpec((NC, RB, 64), lambda i: (0, i, 0)),
                  pl.BlockSpec((1, 128), lambda i: (0, 0))],
        out_specs=pl.BlockSpec((RB, 128), lambda i: (i, 0)),
        out_shape=jax.ShapeDtypeStruct((NP, 128), F32))(
            pa, pb, b.reshape(1, 128))


def _combine16(p, mode):
    rows = p.shape[1]
    pad16 = jnp.eye(16, 128, dtype=F32)

    def body(p_ref, pad_ref, o_ref):
        v = p_ref[0, :, :] + p_ref[1, :, :]
        if mode == "rsqrt":
            v = jnp.where(v > 0, lax.rsqrt(jnp.maximum(v, 1e-30)), 0.0)
        o_ref[...] = jnp.dot(v, pad_ref[...], preferred_element_type=F32)

    return pl.pallas_call(
        body, grid=(rows // RB,),
        in_specs=[pl.BlockSpec((NC, RB, 16), lambda i: (0, i, 0)),
                  pl.BlockSpec((16, 128), lambda i: (0, 0))],
        out_specs=pl.BlockSpec((RB, 128), lambda i: (i, 0)),
        out_shape=jax.ShapeDtypeStruct((rows, 128), F32))(p, pad16)


def _qb_kernel(Spad, Wq, bq, Expand, HmT):
    def body(s_ref, wq_ref, bq_ref, ex_ref, hm_ref, qp_ref, qbt_ref):
        qp = jnp.dot(s_ref[...], wq_ref[...], preferred_element_type=F32)
        qp = qp + bq_ref[...]
        rid = lax.broadcasted_iota(I32, (S1P, 1), 0)
        qp = jnp.where(rid < S1, qp, 0.0)
        qp_ref[...] = qp
        qbt = lax.dot_general(ex_ref[...], qp, (((0,), (0,)), ((), ())),
                              preferred_element_type=F32)
        qbt_ref[...] = qbt * hm_ref[...]

    return pl.pallas_call(
        body,
        out_shape=[jax.ShapeDtypeStruct((S1P, HID), F32),
                   jax.ShapeDtypeStruct((S1 * 8 + 40, HID), F32)],
    )(Spad, Wq, bq.reshape(1, HID), Expand, HmT)


def _scores(Kmat, Qb):
    M = Qb.shape[1]

    def body(k_ref, qb_ref, sc_ref, cm_ref):
        i = pl.program_id(0)
        sc = jnp.dot(k_ref[...], qb_ref[...], preferred_element_type=F32) * ISQ
        sc_ref[...] = sc

        @pl.when(i == 0)
        def _():
            cm_ref[...] = jnp.zeros_like(cm_ref)

        cm_ref[...] = jnp.maximum(cm_ref[...], sc.max(axis=0, keepdims=True))

    return pl.pallas_call(
        body, grid=(NP // RB,),
        in_specs=[pl.BlockSpec((RB, HID), lambda i: (i, 0)),
                  pl.BlockSpec((HID, M), lambda i: (0, 0))],
        out_specs=[pl.BlockSpec((RB, M), lambda i: (i, 0)),
                   pl.BlockSpec((1, M), lambda i: (0, 0))],
        out_shape=[jax.ShapeDtypeStruct((NP, M), F32),
                   jax.ShapeDtypeStruct((1, M), F32)])(Kmat, Qb)


def _ragged_pma1(cum, scores, cmax, V, Qp, Sel, Bm, Wo, bo):
    M = scores.shape[1]

    def body(cum_ref, sc_ref, cm_ref, v_ref, qp_ref, sel_ref, bm_ref,
             wo_ref, bo_ref, o_ref):
        g = pl.program_id(0)
        start = jnp.clip(cum_ref[g], 0, N)
        end = jnp.clip(cum_ref[g + 1], start, N)
        w0 = (start // CH) * CH
        nch = (end - w0 + CH - 1) // CH
        cmax_row = cm_ref[...]

        def erows(k):
            off = pl.multiple_of(w0 + k * CH, CH)
            rows = sc_ref[pl.ds(off, CH), :]
            e = jnp.exp(rows - cmax_row)
            rid = off + lax.broadcasted_iota(I32, (CH, 1), 0)
            m = (rid >= start) & (rid < end)
            return jnp.where(m, e, 0.0)

        def dbody(k, acc):
            return acc + erows(k).sum(axis=0, keepdims=True)

        den = lax.fori_loop(0, nch, dbody, jnp.zeros((1, M), F32))
        den = jnp.where(den == 0, 1.0, den)

        def nbody(k, acc):
            re = erows(k) / den
            rv = v_ref[pl.ds(pl.multiple_of(w0 + k * CH, CH), CH), :]
            return acc + lax.dot_general(re, rv, (((0,), (0,)), ((), ())),
                                         preferred_element_type=F32)

        num = lax.fori_loop(0, nch, nbody, jnp.zeros((M, HID), F32))
        attn = jnp.dot(sel_ref[...], num * bm_ref[...],
                       preferred_element_type=F32)
        O = qp_ref[...] + attn
        o2 = O + jnp.maximum(
            jnp.dot(O, wo_ref[...], preferred_element_type=F32) + bo_ref[...],
            0.0)
        rid = lax.broadcasted_iota(I32, (S1P, 1), 0)
        o_ref[0] = jnp.where(rid < S1, o2, 0.0)

    grid_spec = pltpu.PrefetchScalarGridSpec(
        num_scalar_prefetch=1, grid=(G,),
        in_specs=[pl.BlockSpec((NP, M), lambda g, c: (0, 0)),
                  pl.BlockSpec((1, M), lambda g, c: (0, 0)),
                  pl.BlockSpec((NP, HID), lambda g, c: (0, 0)),
                  pl.BlockSpec((S1P, HID), lambda g, c: (0, 0)),
                  pl.BlockSpec((S1P, M), lambda g, c: (0, 0)),
                  pl.BlockSpec((M, HID), lambda g, c: (0, 0)),
                  pl.BlockSpec((HID, HID), lambda g, c: (0, 0)),
                  pl.BlockSpec((1, HID), lambda g, c: (0, 0))],
        out_specs=pl.BlockSpec((1, S1P, HID), lambda g, c: (g, 0, 0)))
    return pl.pallas_call(
        body, grid_spec=grid_spec,
        out_shape=jax.ShapeDtypeStruct((G, S1P, HID), F32))(
            cum, scores, cmax, V, Qp, Sel, Bm, Wo, bo.reshape(1, HID))


def _sab(X, prm):
    wq, bq = prm["q"]["W"], prm["q"]["b"].reshape(1, HID)
    wk, bk = prm["k"]["W"], prm["k"]["b"].reshape(1, HID)
    wv, bv = prm["v"]["W"], prm["v"]["b"].reshape(1, HID)
    wo, bo = prm["o"]["W"], prm["o"]["b"].reshape(1, HID)
    hd = HID // HEADS

    def body(x_ref, wq_ref, bq_ref, wk_ref, bk_ref, wv_ref, bv_ref,
             wo_ref, bo_ref, o_ref, o_scr):
        X = x_ref[0]
        Q = jnp.dot(X, wq_ref[...], preferred_element_type=F32) + bq_ref[...]
        K = jnp.dot(X, wk_ref[...], preferred_element_type=F32) + bk_ref[...]
        V = jnp.dot(X, wv_ref[...], preferred_element_type=F32) + bv_ref[...]
        kmask = jnp.where(
            lax.broadcasted_iota(I32, (S1P, S1P), 1) < S1, 0.0, -1e9)
        for h in range(HEADS):
            Qh = Q[:, h * hd:(h + 1) * hd]
            Kh = K[:, h * hd:(h + 1) * hd]
            Vh = V[:, h * hd:(h + 1) * hd]
            sc = lax.dot_general(Qh, Kh, (((1,), (1,)), ((), ())),
                                 preferred_element_type=F32) * ISQ + kmask
            m = sc.max(axis=1, keepdims=True)
            e = jnp.exp(sc - m)
            A = e / e.sum(axis=1, keepdims=True)
            o_scr[:, h * hd:(h + 1) * hd] = jnp.dot(
                A, Vh, preferred_element_type=F32)
        O = Q + o_scr[...]
        o_ref[0] = O + jnp.maximum(
            jnp.dot(O, wo_ref[...], preferred_element_type=F32) + bo_ref[...],
            0.0)

    cst = lambda i: (0, 0)
    return pl.pallas_call(
        body, grid=(G,),
        in_specs=[pl.BlockSpec((1, S1P, HID), lambda i: (i, 0, 0)),
                  pl.BlockSpec((HID, HID), cst), pl.BlockSpec((1, HID), cst),
                  pl.BlockSpec((HID, HID), cst), pl.BlockSpec((1, HID), cst),
                  pl.BlockSpec((HID, HID), cst), pl.BlockSpec((1, HID), cst),
                  pl.BlockSpec((HID, HID), cst), pl.BlockSpec((1, HID), cst)],
        out_specs=pl.BlockSpec((1, S1P, HID), lambda i: (i, 0, 0)),
        out_shape=jax.ShapeDtypeStruct((G, S1P, HID), F32),
        scratch_shapes=[pltpu.VMEM((S1P, HID), F32)],
    )(X, wq, bq, wk, bk, wv, bv, wo, bo)


def _pma2_head(X, s2, prm, lin2, fc0, outp):
    wq, bq = prm["q"]["W"], prm["q"]["b"].reshape(1, HID)
    wk, bk = prm["k"]["W"], prm["k"]["b"].reshape(1, HID)
    wv, bv = prm["v"]["W"], prm["v"]["b"].reshape(1, HID)
    wo, bo = prm["o"]["W"], prm["o"]["b"].reshape(1, HID)
    wl2, bl2 = lin2["W"], lin2["b"].reshape(1, HID)
    wf0, bf0 = fc0["W"], fc0["b"].reshape(1, HID)
    wout = jnp.zeros((HID, 128), F32).at[:, 0].set(outp["W"][:, 0])
    bout = jnp.zeros((1, 128), F32).at[0, 0].set(outp["b"][0])
    hd = HID // HEADS

    def body(x_ref, s2_ref, wq_ref, bq_ref, wk_ref, bk_ref, wv_ref, bv_ref,
             wo_ref, bo_ref, wl2_ref, bl2_ref, wf0_ref, bf0_ref,
             wout_ref, bout_ref, o_ref, o_scr):
        X = x_ref[0]
        Qp = jnp.dot(s2_ref[...], wq_ref[...],
                     preferred_element_type=F32) + bq_ref[...]
        K = jnp.dot(X, wk_ref[...], preferred_element_type=F32) + bk_ref[...]
        V = jnp.dot(X, wv_ref[...], preferred_element_type=F32) + bv_ref[...]
        kmask = jnp.where(
            lax.broadcasted_iota(I32, (1, S1P), 1) < S1, 0.0, -1e9)
        for h in range(HEADS):
            Qh = Qp[:, h * hd:(h + 1) * hd]
            Kh = K[:, h * hd:(h + 1) * hd]
            Vh = V[:, h * hd:(h + 1) * hd]
            sc = lax.dot_general(Qh, Kh, (((1,), (1,)), ((), ())),
                                 preferred_element_type=F32) * ISQ + kmask
            m = sc.max(axis=1, keepdims=True)
            e = jnp.exp(sc - m)
            A = e / e.sum(axis=1, keepdims=True)
            o_scr[:, h * hd:(h + 1) * hd] = jnp.dot(
                A, Vh, preferred_element_type=F32)
        O = Qp + o_scr[...]
        O = O + jnp.maximum(
            jnp.dot(O, wo_ref[...], preferred_element_type=F32) + bo_ref[...],
            0.0)
        res = jnp.dot(O, wl2_ref[...], preferred_element_type=F32) + bl2_ref[...]
        res = jnp.maximum(
            jnp.dot(res, wf0_ref[...], preferred_element_type=F32) + bf0_ref[...],
            0.0)
        o_ref[0] = jnp.dot(res, wout_ref[...],
                           preferred_element_type=F32) + bout_ref[...]

    cst = lambda i: (0, 0)
    return pl.pallas_call(
        body, grid=(G,),
        in_specs=[pl.BlockSpec((1, S1P, HID), lambda i: (i, 0, 0)),
                  pl.BlockSpec((1, HID), cst),
                  pl.BlockSpec((HID, HID), cst), pl.BlockSpec((1, HID), cst),
                  pl.BlockSpec((HID, HID), cst), pl.BlockSpec((1, HID), cst),
                  pl.BlockSpec((HID, HID), cst), pl.BlockSpec((1, HID), cst),
                  pl.BlockSpec((HID, HID), cst), pl.BlockSpec((1, HID), cst),
                  pl.BlockSpec((HID, HID), cst), pl.BlockSpec((1, HID), cst),
                  pl.BlockSpec((HID, HID), cst), pl.BlockSpec((1, HID), cst),
                  pl.BlockSpec((HID, 128), cst), pl.BlockSpec((1, 128), cst)],
        out_specs=pl.BlockSpec((1, 1, 128), lambda i: (i, 0, 0)),
        out_shape=jax.ShapeDtypeStruct((G, 1, 128), F32),
        scratch_shapes=[pltpu.VMEM((1, HID), F32)],
    )(X, s2, wq, bq, wk, bk, wv, bv, wo, bo, wl2, bl2, wf0, bf0, wout, bout)


def _pad_rows(a, rows, value=0.0):
    return jnp.pad(a, ((0, rows - a.shape[0]),) + ((0, 0),) * (a.ndim - 1),
                   constant_values=value)


def kernel(x, edge_index, edge_attr, batch, params):
    del edge_attr
    idt = edge_index.dtype
    loop = jnp.arange(N, dtype=idt)
    s_all = jnp.concatenate([edge_index[0], loop])
    d_all = jnp.concatenate([edge_index[1], loop])
    s_pad = jnp.pad(s_all, (0, EP - EL), constant_values=N).astype(I32)
    d_pad = jnp.pad(d_all, (0, EP - EL), constant_values=N).astype(I32)
    batch_pad = jnp.pad(batch, (0, BP - N), constant_values=G).astype(I32)
    xp = _pad_rows(x, NP)

    zeros16 = jnp.zeros((NP, 16), F32)
    zeros64 = jnp.zeros((NP, 64), F32)
    zeros_gt = jnp.zeros((GT_ROWS, 16), F32)
    ones_chunk = jnp.ones((CHUNK, 16), F32)
    ones_ep16 = jnp.ones((EP, 16), F32)

    gather = _make_gather(NP, EP)
    scat16 = _make_scatter(NP, EP, 16)
    scat64 = _make_scatter(NP, EP, 64)

    cnt_p = _make_count(GT_ROWS, BP)(batch_pad, zeros_gt, ones_chunk)
    counts = (cnt_p[0, :G, 0] + cnt_p[1, :G, 0]).astype(I32)
    cum = jnp.concatenate([jnp.zeros((1,), I32),
                           jnp.cumsum(counts)]).astype(I32)

    deg_p = scat16(ones_ep16, d_pad, zeros16)
    dinv = _combine16(deg_p, mode="rsqrt")

    eye8 = jnp.eye(HEADS, dtype=F32)
    h = xp
    for i in range(3):
        prm = params["gat"][i]
        AsM = jnp.einsum("hd,hg->hdg", prm["att_src"], eye8).reshape(D_XL, HEADS)
        AsM = jnp.pad(AsM, ((0, 0), (0, 120)))
        AdM = jnp.einsum("hd,hg->hdg", prm["att_dst"], eye8).reshape(D_XL, HEADS)
        AdM = jnp.pad(AdM, ((0, 0), (0, 120)))
        W = prm["W"]
        As, Ad, Ms, Md = _gat_prep(h, W @ AsM, W @ AdM)
        als = gather(As, s_pad)
        ald = gather(Ad, d_pad)
        ex16 = _edge_ex(als, ald, Ms, Md)
        den_p = scat16(ex16, d_pad, zeros16)
        den = _combine16(den_p, mode="add")
        deng = gather(den, d_pad)
        att16 = _edge_div(ex16, deng)
        xgs = [gather(_mm(h, W[:, hh * 128:(hh + 1) * 128]), s_pad)
               for hh in range(HEADS)]
        mlo, mhi = _edge_msg(att16, xgs, HEADS)
        plo = scat64(mlo, d_pad, zeros64)
        phi = scat64(mhi, d_pad, zeros64)
        h = _combine128(plo, phi, prm["b"], 1.0 / HEADS,
                        relu=True, mask_rows=True)

    x1 = _mm(h, params["lin1"]["W"], params["lin1"]["b"])
    gs = gather(dinv, s_pad)
    gd = gather(dinv, d_pad)
    norm16 = _edge_mul(gs, gd)
    kv = {}
    for nm in ("k", "v"):
        xli = _mm(x1, params["pma1"][nm]["W"])
        xg = gather(xli, s_pad)
        mlo, mhi = _edge_msg(norm16, [xg], 1)
        plo = scat64(mlo, d_pad, zeros64)
        phi = scat64(mhi, d_pad, zeros64)
        kv[nm] = _combine128(plo, phi, params["pma1"][nm]["b"], 1.0,
                             relu=False, mask_rows=True)

    M640 = S1 * 8 + 40
    Spad = _pad_rows(params["pma1_S"], S1P)
    ex_np = np.zeros((S1P, M640), np.float32)
    for s in range(S1):
        for hh in range(HEADS):
            ex_np[s, s * 8 + hh] = 1.0
    Expand = jnp.asarray(ex_np)
    hm_np = np.zeros((M640, HID), np.float32)
    for c in range(M640):
        hh = c % 8
        hm_np[c, hh * 16:(hh + 1) * 16] = 1.0
    HmT = jnp.asarray(hm_np)
    sel_np = np.zeros((S1P, M640), np.float32)
    for s in range(S1P):
        for hh in range(HEADS):
            sel_np[s, s * 8 + hh] = 1.0
    Sel = jnp.asarray(sel_np)

    Qp, QbT = _qb_kernel(Spad, params["pma1"]["q"]["W"],
                         params["pma1"]["q"]["b"], Expand, HmT)
    scores, cmax = _scores(kv["k"], QbT.T)
    pooled = _ragged_pma1(cum, scores, cmax, kv["v"], Qp, Sel, HmT,
                          params["pma1"]["o"]["W"], params["pma1"]["o"]["b"])

    sab_out = _sab(pooled, params["sab"])
    yy = _pma2_head(sab_out, params["pma2_S"], params["pma2"],
                    params["lin2"], params["fc0"], params["out"])
    return yy[:, 0, :1]
